# Initial kernel scaffold; baseline (speedup 1.0000x reference)
import jax
import jax.numpy as jnp
from jax.experimental import pallas as pl


def kernel(stud_ids, ques_ids, responses, params):
    raise NotImplementedError("write your pallas kernel here")


# speedup vs baseline: 2.0570x; 2.0570x over previous
import functools

import jax
import jax.numpy as jnp
from jax import lax
from jax.experimental import pallas as pl
from jax.experimental.pallas import tpu as pltpu
from jax.experimental.pallas import tpu_sc as plsc

N_STUDENTS = 100000
BATCH = 4096
NSUB = 16
LANES = 16
ROWS_PER_TILE = 2
ROW = 128
CHUNKS = ROW // LANES

_LN2 = 0.6931471805599453
_INV_B = 1.0 / BATCH


def _softplus16(d):
    x = jnp.float32(1.0) + jnp.exp(d)
    xi = lax.bitcast_convert_type(x, jnp.int32)
    e = lax.shift_right_arithmetic(xi, jnp.int32(23)) - jnp.int32(127)
    m_bits = lax.bitwise_or(
        lax.bitwise_and(xi, jnp.int32(0x007FFFFF)), jnp.int32(0x3F800000)
    )
    m = lax.bitcast_convert_type(m_bits, jnp.float32)
    t = (m - jnp.float32(1.0)) / (m + jnp.float32(1.0))
    t2 = t * t
    p = jnp.float32(1.0 / 9.0)
    p = p * t2 + jnp.float32(1.0 / 7.0)
    p = p * t2 + jnp.float32(1.0 / 5.0)
    p = p * t2 + jnp.float32(1.0 / 3.0)
    p = p * t2 + jnp.float32(1.0)
    log_m = jnp.float32(2.0) * t * p
    return e.astype(jnp.float32) * jnp.float32(_LN2) + log_m


@functools.partial(
    pl.kernel,
    out_type=(
        jax.ShapeDtypeStruct((LANES,), jnp.float32),
        jax.ShapeDtypeStruct((NSUB, 4, LANES), jnp.float32),
    ),
    mesh=plsc.VectorSubcoreMesh(core_axis_name="c", subcore_axis_name="s"),
    scratch_types=[
        pltpu.VMEM((ROWS_PER_TILE, ROW), jnp.int32),
        pltpu.VMEM((ROWS_PER_TILE, ROW), jnp.int32),
        pltpu.VMEM((ROWS_PER_TILE, ROW), jnp.float32),
        pltpu.VMEM((ROWS_PER_TILE, ROW), jnp.float32),
        pltpu.VMEM((ROWS_PER_TILE, ROW), jnp.float32),
        pltpu.VMEM((4, LANES), jnp.float32),
        pltpu.VMEM((NSUB, 4, LANES), jnp.float32),
        pltpu.VMEM((LANES,), jnp.float32),
        pltpu.SemaphoreType.DMA,
    ],
)
def _rasch_sc(params_hbm, sids_hbm, qids_hbm, resp_hbm, out_hbm, stage_hbm,
              sidx, qidx, srows, qrows, resp, accbuf, gath, outbuf, sem):
    cid = lax.axis_index("c")
    sid = lax.axis_index("s")

    @pl.when(cid == 0)
    def _():
        pltpu.sync_copy(sids_hbm.at[sid], sidx)
        pltpu.sync_copy(qids_hbm.at[sid], qidx)
        pltpu.sync_copy(resp_hbm.at[sid], resp)

        for r in range(ROWS_PER_TILE):
            for c in range(CHUNKS):
                sl = pl.ds(c * LANES, LANES)
                qidx[r, sl] = qidx[r, sl] + jnp.int32(N_STUDENTS)

        copies = []
        for r in range(ROWS_PER_TILE):
            copies.append(
                pltpu.async_copy(params_hbm.at[sidx.at[r]], srows.at[r], sem))
            copies.append(
                pltpu.async_copy(params_hbm.at[qidx.at[r]], qrows.at[r], sem))
        for cpy in copies:
            cpy.wait()

        acc_sp = jnp.zeros((LANES,), jnp.float32)
        acc_d = jnp.zeros((LANES,), jnp.float32)
        acc_r = jnp.zeros((LANES,), jnp.float32)
        for r in range(ROWS_PER_TILE):
            for c in range(CHUNKS):
                sl = pl.ds(c * LANES, LANES)
                d = srows[r, sl] - qrows[r, sl]
                acc_sp = acc_sp + _softplus16(d)
                acc_d = acc_d + d
                acc_r = acc_r + resp[r, sl]

        accbuf[0, :] = acc_sp
        accbuf[1, :] = acc_d
        accbuf[2, :] = acc_r
        accbuf[3, :] = jnp.zeros((LANES,), jnp.float32)

        pltpu.sync_copy(accbuf, stage_hbm.at[sid])
        plsc.subcore_barrier()

        @pl.when(sid == 0)
        def _():
            pltpu.sync_copy(stage_hbm, gath)
            tsp = jnp.zeros((LANES,), jnp.float32)
            td = jnp.zeros((LANES,), jnp.float32)
            tr = jnp.zeros((LANES,), jnp.float32)
            for i in range(NSUB):
                tsp = tsp + gath[i, 0, :]
                td = td + gath[i, 1, :]
                tr = tr + gath[i, 2, :]
            ssp = jnp.float32(0.0)
            sd = jnp.float32(0.0)
            sr = jnp.float32(0.0)
            for k in range(LANES):
                ssp = ssp + tsp[k]
                sd = sd + td[k]
                sr = sr + tr[k]
            res = ssp * jnp.float32(_INV_B) - (
                sr * jnp.float32(_INV_B)) * (sd * jnp.float32(_INV_B))
            outbuf[...] = res * jnp.ones((LANES,), jnp.float32)
            pltpu.sync_copy(outbuf, out_hbm)


def kernel(stud_ids, ques_ids, responses, params):
    sids = stud_ids.astype(jnp.int32).reshape(NSUB, ROWS_PER_TILE, ROW)
    qids = ques_ids.astype(jnp.int32).reshape(NSUB, ROWS_PER_TILE, ROW)
    resp = responses.astype(jnp.float32).reshape(NSUB, ROWS_PER_TILE, ROW)
    table = params.reshape(-1)
    out, _ = _rasch_sc(table, sids, qids, resp)
    return out[0]


# speedup vs baseline: 2.2329x; 1.0855x over previous
import functools

import jax
import jax.numpy as jnp
from jax import lax
from jax.experimental import pallas as pl
from jax.experimental.pallas import tpu as pltpu
from jax.experimental.pallas import tpu_sc as plsc

N_STUDENTS = 100000
N_QUESTIONS = 100000
N_PARAMS = N_STUDENTS + N_QUESTIONS
BATCH = 4096
NSUB = 16
LANES = 16
ROWS_PER_TILE = 2
ROW = 128
CHUNKS = ROW // LANES

_LN2 = 0.6931471805599453
_INV_B = 1.0 / BATCH


def _softplus16(d):
    x = jnp.float32(1.0) + jnp.exp(d)
    xi = lax.bitcast_convert_type(x, jnp.int32)
    e = lax.shift_right_arithmetic(xi, jnp.int32(23)) - jnp.int32(127)
    m_bits = lax.bitwise_or(
        lax.bitwise_and(xi, jnp.int32(0x007FFFFF)), jnp.int32(0x3F800000)
    )
    m = lax.bitcast_convert_type(m_bits, jnp.float32)
    t = (m - jnp.float32(1.0)) / (m + jnp.float32(1.0))
    t2 = t * t
    p = jnp.float32(1.0 / 9.0)
    p = p * t2 + jnp.float32(1.0 / 7.0)
    p = p * t2 + jnp.float32(1.0 / 5.0)
    p = p * t2 + jnp.float32(1.0 / 3.0)
    p = p * t2 + jnp.float32(1.0)
    log_m = jnp.float32(2.0) * t * p
    return e.astype(jnp.float32) * jnp.float32(_LN2) + log_m


@functools.partial(
    pl.kernel,
    out_type=(
        jax.ShapeDtypeStruct((LANES,), jnp.float32),
        jax.ShapeDtypeStruct((NSUB, 4, LANES), jnp.float32),
    ),
    mesh=plsc.VectorSubcoreMesh(core_axis_name="c", subcore_axis_name="s",
                                num_cores=1),
    scratch_types=[
        pltpu.VMEM((ROWS_PER_TILE, ROW), jnp.int32),
        pltpu.VMEM((ROWS_PER_TILE, ROW), jnp.int32),
        pltpu.VMEM((ROWS_PER_TILE, ROW), jnp.float32),
        pltpu.VMEM((ROWS_PER_TILE, ROW), jnp.float32),
        pltpu.VMEM((ROWS_PER_TILE, ROW), jnp.float32),
        pltpu.VMEM((4, LANES), jnp.float32),
        pltpu.VMEM((NSUB, 4, LANES), jnp.float32),
        pltpu.VMEM((LANES,), jnp.float32),
        pltpu.SemaphoreType.DMA,
        pltpu.SemaphoreType.DMA,
    ],
)
def _rasch_sc(params_hbm, sids_hbm, qids_hbm, resp_hbm,
              out_hbm, stage_hbm,
              sidx, qidx, srows, qrows, resp, accbuf, gath, outbuf,
              sem, isem):
    cid = lax.axis_index("c")
    sid = lax.axis_index("s")

    @pl.when(cid == 0)
    def _():
        c1 = pltpu.async_copy(sids_hbm.at[sid], sidx, isem)
        c2 = pltpu.async_copy(qids_hbm.at[sid], qidx, isem)
        c3 = pltpu.async_copy(resp_hbm.at[sid], resp, isem)
        c1.wait()
        c2.wait()

        for r in range(ROWS_PER_TILE):
            for c in range(CHUNKS):
                sl = pl.ds(c * LANES, LANES)
                qidx[r, sl] = qidx[r, sl] + jnp.int32(N_STUDENTS)

        copies = []
        for r in range(ROWS_PER_TILE):
            copies.append(
                pltpu.async_copy(params_hbm.at[sidx.at[r]], srows.at[r], sem))
            copies.append(
                pltpu.async_copy(params_hbm.at[qidx.at[r]], qrows.at[r], sem))
        c3.wait()

        acc_sp = jnp.zeros((LANES,), jnp.float32)
        acc_d = jnp.zeros((LANES,), jnp.float32)
        acc_r = jnp.zeros((LANES,), jnp.float32)
        for r in range(ROWS_PER_TILE):
            copies[2 * r].wait()
            copies[2 * r + 1].wait()
            for c in range(CHUNKS):
                sl = pl.ds(c * LANES, LANES)
                d = srows[r, sl] - qrows[r, sl]
                acc_sp = acc_sp + _softplus16(d)
                acc_d = acc_d + d
                acc_r = acc_r + resp[r, sl]

        accbuf[0, :] = acc_sp
        accbuf[1, :] = acc_d
        accbuf[2, :] = acc_r
        accbuf[3, :] = jnp.zeros((LANES,), jnp.float32)

        pltpu.sync_copy(accbuf, stage_hbm.at[sid])
        plsc.subcore_barrier()

        @pl.when(sid == 0)
        def _():
            pltpu.sync_copy(stage_hbm, gath)
            tsp = jnp.zeros((LANES,), jnp.float32)
            td = jnp.zeros((LANES,), jnp.float32)
            tr = jnp.zeros((LANES,), jnp.float32)
            for i in range(NSUB):
                tsp = tsp + gath[i, 0, :]
                td = td + gath[i, 1, :]
                tr = tr + gath[i, 2, :]
            ssp = jnp.float32(0.0)
            sd = jnp.float32(0.0)
            sr = jnp.float32(0.0)
            for k in range(LANES):
                ssp = ssp + tsp[k]
                sd = sd + td[k]
                sr = sr + tr[k]
            res = ssp * jnp.float32(_INV_B) - (
                sr * jnp.float32(_INV_B)) * (sd * jnp.float32(_INV_B))
            outbuf[...] = res * jnp.ones((LANES,), jnp.float32)
            pltpu.sync_copy(outbuf, out_hbm)


def kernel(stud_ids, ques_ids, responses, params):
    sids = stud_ids.astype(jnp.int32).reshape(NSUB, ROWS_PER_TILE, ROW)
    qids = ques_ids.astype(jnp.int32).reshape(NSUB, ROWS_PER_TILE, ROW)
    resp = responses.astype(jnp.float32).reshape(NSUB, ROWS_PER_TILE, ROW)
    table = params.reshape(-1)
    out, _ = _rasch_sc(table, sids, qids, resp)
    return out[0]


# speedup vs baseline: 2.2401x; 1.0032x over previous
import functools

import jax
import jax.numpy as jnp
from jax import lax
from jax.experimental import pallas as pl
from jax.experimental.pallas import tpu as pltpu
from jax.experimental.pallas import tpu_sc as plsc

N_STUDENTS = 100000
N_QUESTIONS = 100000
N_PARAMS = N_STUDENTS + N_QUESTIONS
BATCH = 4096
NSUB = 16
LANES = 16
ROWS_PER_TILE = 2
ROW = 128
CHUNKS = ROW // LANES

_LN2 = 0.6931471805599453
_INV_B = 1.0 / BATCH


def _softplus16(d):
    x = jnp.float32(1.0) + jnp.exp(d)
    xi = lax.bitcast_convert_type(x, jnp.int32)
    e = lax.shift_right_arithmetic(xi, jnp.int32(23)) - jnp.int32(127)
    m_bits = lax.bitwise_or(
        lax.bitwise_and(xi, jnp.int32(0x007FFFFF)), jnp.int32(0x3F800000)
    )
    m = lax.bitcast_convert_type(m_bits, jnp.float32)
    t = (m - jnp.float32(1.0)) / (m + jnp.float32(1.0))
    t2 = t * t
    p = jnp.float32(1.0 / 9.0)
    p = p * t2 + jnp.float32(1.0 / 7.0)
    p = p * t2 + jnp.float32(1.0 / 5.0)
    p = p * t2 + jnp.float32(1.0 / 3.0)
    p = p * t2 + jnp.float32(1.0)
    log_m = jnp.float32(2.0) * t * p
    return e.astype(jnp.float32) * jnp.float32(_LN2) + log_m


@functools.partial(
    pl.kernel,
    out_type=jax.ShapeDtypeStruct((LANES,), jnp.float32),
    mesh=plsc.VectorSubcoreMesh(core_axis_name="c", subcore_axis_name="s",
                                num_cores=1),
    scratch_types=[
        pltpu.VMEM((ROWS_PER_TILE, ROW), jnp.int32),
        pltpu.VMEM((ROWS_PER_TILE, ROW), jnp.int32),
        pltpu.VMEM((ROWS_PER_TILE, ROW), jnp.float32),
        pltpu.VMEM((ROWS_PER_TILE, ROW), jnp.float32),
        pltpu.VMEM((ROWS_PER_TILE, ROW), jnp.float32),
        pltpu.VMEM((4, LANES), jnp.float32),
        pltpu.VMEM((NSUB, 4, LANES), jnp.float32),
        pltpu.HBM((NSUB, 4, LANES), jnp.float32),
        pltpu.VMEM((LANES,), jnp.float32),
        pltpu.SemaphoreType.DMA,
        pltpu.SemaphoreType.DMA,
    ],
)
def _rasch_sc(params_hbm, sids_hbm, qids_hbm, resp_hbm,
              out_hbm,
              sidx, qidx, srows, qrows, resp, accbuf, gath, stage_hbm,
              outbuf, sem, isem):
    cid = lax.axis_index("c")
    sid = lax.axis_index("s")

    @pl.when(cid == 0)
    def _():
        c1 = pltpu.async_copy(sids_hbm.at[sid], sidx, isem)
        c2 = pltpu.async_copy(qids_hbm.at[sid], qidx, isem)
        c3 = pltpu.async_copy(resp_hbm.at[sid], resp, isem)
        c1.wait()
        c2.wait()

        for r in range(ROWS_PER_TILE):
            for c in range(CHUNKS):
                sl = pl.ds(c * LANES, LANES)
                qidx[r, sl] = qidx[r, sl] + jnp.int32(N_STUDENTS)

        copies = []
        for r in range(ROWS_PER_TILE):
            copies.append(
                pltpu.async_copy(params_hbm.at[sidx.at[r]], srows.at[r], sem))
            copies.append(
                pltpu.async_copy(params_hbm.at[qidx.at[r]], qrows.at[r], sem))
        c3.wait()

        acc_sp = jnp.zeros((LANES,), jnp.float32)
        acc_d = jnp.zeros((LANES,), jnp.float32)
        acc_r = jnp.zeros((LANES,), jnp.float32)
        for r in range(ROWS_PER_TILE):
            copies[2 * r].wait()
            copies[2 * r + 1].wait()
            for c in range(CHUNKS):
                sl = pl.ds(c * LANES, LANES)
                d = srows[r, sl] - qrows[r, sl]
                acc_sp = acc_sp + _softplus16(d)
                acc_d = acc_d + d
                acc_r = acc_r + resp[r, sl]

        accbuf[0, :] = acc_sp
        accbuf[1, :] = acc_d
        accbuf[2, :] = acc_r
        accbuf[3, :] = jnp.zeros((LANES,), jnp.float32)

        pltpu.sync_copy(accbuf, stage_hbm.at[sid])
        plsc.subcore_barrier()

        @pl.when(sid == 0)
        def _():
            pltpu.sync_copy(stage_hbm, gath)
            tsp = jnp.zeros((LANES,), jnp.float32)
            td = jnp.zeros((LANES,), jnp.float32)
            tr = jnp.zeros((LANES,), jnp.float32)
            for i in range(NSUB):
                tsp = tsp + gath[i, 0, :]
                td = td + gath[i, 1, :]
                tr = tr + gath[i, 2, :]
            ssp = jnp.float32(0.0)
            sd = jnp.float32(0.0)
            sr = jnp.float32(0.0)
            for k in range(LANES):
                ssp = ssp + tsp[k]
                sd = sd + td[k]
                sr = sr + tr[k]
            res = ssp * jnp.float32(_INV_B) - (
                sr * jnp.float32(_INV_B)) * (sd * jnp.float32(_INV_B))
            outbuf[...] = res * jnp.ones((LANES,), jnp.float32)
            pltpu.sync_copy(outbuf, out_hbm)


def kernel(stud_ids, ques_ids, responses, params):
    sids = stud_ids.astype(jnp.int32).reshape(NSUB, ROWS_PER_TILE, ROW)
    qids = ques_ids.astype(jnp.int32).reshape(NSUB, ROWS_PER_TILE, ROW)
    resp = responses.astype(jnp.float32).reshape(NSUB, ROWS_PER_TILE, ROW)
    table = params.reshape(-1)
    out = _rasch_sc(table, sids, qids, resp)
    return out[0]
